# baseline (speedup 1.0000x reference)
import functools

import jax
import jax.numpy as jnp
from jax.experimental import pallas as pl
from jax.experimental.pallas import tpu as pltpu

_TC = 512
_K = 16
_PAD = 128


def _pass1_body(cpb, x_ref, w_ref, et_ref, s_ref):
    i = pl.program_id(0)
    xb = x_ref[...]
    z = jax.lax.dot_general(
        xb, w_ref[...],
        (((1,), (0,)), ((), ())),
        preferred_element_type=jnp.float32,
    )
    et_ref[...] = jnp.tanh(z)
    part = jnp.sum(xb, axis=0, keepdims=True)[:, None, :]

    @pl.when(i % cpb == 0)
    def _():
        s_ref[...] = part

    @pl.when(i % cpb != 0)
    def _():
        s_ref[...] += part


def _topk_body(et_ref, s_ref, idx_ref, wz_ref, sz_ref):
    et = et_ref[...]
    b, t = et.shape
    iota = jax.lax.broadcasted_iota(jnp.int32, (b, t), 1)
    k_iota = jax.lax.broadcasted_iota(jnp.int32, (b, _PAD), 1)
    cur = et
    vals = jnp.full((b, _PAD), -2.0, jnp.float32)
    idxs = jnp.zeros((b, _PAD), jnp.int32)
    for k in range(_K):
        v = jnp.max(cur, axis=1, keepdims=True)
        am = jnp.min(jnp.where(cur == v, iota, t), axis=1, keepdims=True)
        vals = jnp.where(k_iota == k, v, vals)
        idxs = jnp.where(k_iota == k, am, idxs)
        cur = jnp.where(iota == am, -2.0, cur)
    thresh = jnp.sum(jnp.where(k_iota == 4, vals, 0.0), axis=1, keepdims=True)
    w = jnp.where(vals >= thresh, jnp.exp(vals) - 1.0, 0.0)
    zden = t + jnp.sum(w, axis=1, keepdims=True)
    idx_ref[...] = idxs
    wz_ref[...] = w / zden
    sz_ref[...] = s_ref[...] / zden


def _tail_body(et_ref, s_ref, x_hbm, out_ref, iscr, wscr, rows, sem):
    et = et_ref[...]
    b, t = et.shape
    iota = jax.lax.broadcasted_iota(jnp.int32, (b, t), 1)
    k_iota = jax.lax.broadcasted_iota(jnp.int32, (b, _PAD), 1)
    cur = et
    vals = jnp.full((b, _PAD), -2.0, jnp.float32)
    idxs = jnp.zeros((b, _PAD), jnp.int32)
    for k in range(_K):
        v = jnp.max(cur, axis=1, keepdims=True)
        am = jnp.min(jnp.where(cur == v, iota, t), axis=1, keepdims=True)
        vals = jnp.where(k_iota == k, v, vals)
        idxs = jnp.where(k_iota == k, am, idxs)
        cur = jnp.where(iota == am, -2.0, cur)
    thresh = jnp.sum(jnp.where(k_iota == 4, vals, 0.0), axis=1, keepdims=True)
    w = jnp.where(vals >= thresh, jnp.exp(vals) - 1.0, 0.0)
    zden = t + jnp.sum(w, axis=1, keepdims=True)
    iscr[...] = idxs
    wscr[...] = jnp.where(k_iota == _K, zden, w)
    for bb in range(b):
        for k in range(_K):
            pltpu.make_async_copy(
                x_hbm.at[bb, pl.ds(iscr[bb, k], 1), :],
                rows.at[pl.ds(bb * _K + k, 1), :], sem).start()
    for bb in range(b):
        for k in range(_K):
            pltpu.make_async_copy(
                x_hbm.at[bb, pl.ds(iscr[bb, k], 1), :],
                rows.at[pl.ds(bb * _K + k, 1), :], sem).wait()
    for bb in range(b):
        acc = s_ref[pl.ds(bb, 1), :]
        for k in range(_K):
            acc = acc + wscr[bb, k] * rows[pl.ds(bb * _K + k, 1), :]
        out_ref[pl.ds(bb, 1), :] = acc / wscr[bb, _K]


def kernel(x, W):
    B, T, D = x.shape
    n_chunks = T // _TC

    tcr = B * _TC
    cpb = T // tcr
    et2, S3 = pl.pallas_call(
        functools.partial(_pass1_body, cpb),
        grid=(B * cpb,),
        in_specs=[
            pl.BlockSpec((tcr, D), lambda i: (i, 0)),
            pl.BlockSpec((D, 1), lambda i: (0, 0)),
        ],
        out_specs=[
            pl.BlockSpec((tcr, 1), lambda i: (i, 0)),
            pl.BlockSpec((1, 1, D), lambda i: (i // cpb, 0, 0)),
        ],
        out_shape=[
            jax.ShapeDtypeStruct((B * T, 1), jnp.float32),
            jax.ShapeDtypeStruct((B, 1, D), jnp.float32),
        ],
    )(x.reshape(B * T, D), W)
    et = et2.reshape(B, T)
    S = S3.reshape(B, D)

    out = pl.pallas_call(
        _tail_body,
        in_specs=[
            pl.BlockSpec((B, T), lambda: (0, 0)),
            pl.BlockSpec((B, D), lambda: (0, 0)),
            pl.BlockSpec(memory_space=pl.ANY),
        ],
        out_specs=pl.BlockSpec((B, D), lambda: (0, 0)),
        out_shape=jax.ShapeDtypeStruct((B, D), jnp.float32),
        scratch_shapes=[
            pltpu.VMEM((B, _PAD), jnp.int32),
            pltpu.VMEM((B, _PAD), jnp.float32),
            pltpu.VMEM((B * _K, D), jnp.float32),
            pltpu.SemaphoreType.DMA,
        ],
    )(et, S, x)

    return out


# speedup vs baseline: 1.3883x; 1.3883x over previous
import functools

import jax
import jax.numpy as jnp
from jax.experimental import pallas as pl
from jax.experimental.pallas import tpu as pltpu

_TC = 512
_K = 8
_PAD = 128


def _pass1_body(x_ref, w_ref, et_ref, s_ref):
    i = pl.program_id(0)
    xb = x_ref[...]
    b, tc, d = xb.shape
    z = jax.lax.dot_general(
        xb.reshape(b * tc, d), w_ref[...],
        (((1,), (0,)), ((), ())),
        preferred_element_type=jnp.float32,
    )
    et_ref[...] = jnp.tanh(z).reshape(b, tc)
    part = jnp.sum(xb, axis=1)

    @pl.when(i == 0)
    def _():
        s_ref[...] = part

    @pl.when(i > 0)
    def _():
        s_ref[...] += part


def _topk_body(et_ref, s_ref, idx_ref, wz_ref, sz_ref):
    et = et_ref[...]
    b, t = et.shape
    iota = jax.lax.broadcasted_iota(jnp.int32, (b, t), 1)
    k_iota = jax.lax.broadcasted_iota(jnp.int32, (b, _PAD), 1)
    cur = et
    vals = jnp.full((b, _PAD), -2.0, jnp.float32)
    idxs = jnp.zeros((b, _PAD), jnp.int32)
    for k in range(_K):
        v = jnp.max(cur, axis=1, keepdims=True)
        am = jnp.min(jnp.where(cur == v, iota, t), axis=1, keepdims=True)
        vals = jnp.where(k_iota == k, v, vals)
        idxs = jnp.where(k_iota == k, am, idxs)
        cur = jnp.where(iota == am, -2.0, cur)
    thresh = jnp.sum(jnp.where(k_iota == 4, vals, 0.0), axis=1, keepdims=True)
    w = jnp.where(vals >= thresh, jnp.exp(vals) - 1.0, 0.0)
    zden = t + jnp.sum(w, axis=1, keepdims=True)
    idx_ref[...] = idxs
    wz_ref[...] = w / zden
    sz_ref[...] = s_ref[...] / zden


def _tail_body(et_ref, s_ref, x_hbm, out_ref, iscr, wscr, rows, sem):
    et = et_ref[...]
    b, t = et.shape
    iota = jax.lax.broadcasted_iota(jnp.int32, (b, t), 1)
    k_iota = jax.lax.broadcasted_iota(jnp.int32, (b, _PAD), 1)
    cur = et
    vals = jnp.full((b, _PAD), -2.0, jnp.float32)
    idxs = jnp.zeros((b, _PAD), jnp.int32)
    for k in range(_K):
        v = jnp.max(cur, axis=1, keepdims=True)
        am = jnp.min(jnp.where(cur == v, iota, t), axis=1, keepdims=True)
        vals = jnp.where(k_iota == k, v, vals)
        idxs = jnp.where(k_iota == k, am, idxs)
        cur = jnp.where(iota == am, -2.0, cur)
    thresh = jnp.sum(jnp.where(k_iota == 4, vals, 0.0), axis=1, keepdims=True)
    w = jnp.where(vals >= thresh, jnp.exp(vals) - 1.0, 0.0)
    zden = t + jnp.sum(w, axis=1, keepdims=True)
    iscr[...] = idxs
    wscr[...] = jnp.where(k_iota == _K, zden, w)
    for bb in range(b):
        for k in range(_K):
            pltpu.make_async_copy(
                x_hbm.at[bb, pl.ds(iscr[bb, k], 1), :],
                rows.at[pl.ds(bb * _K + k, 1), :], sem).start()
    for bb in range(b):
        for k in range(_K):
            pltpu.make_async_copy(
                x_hbm.at[bb, pl.ds(iscr[bb, k], 1), :],
                rows.at[pl.ds(bb * _K + k, 1), :], sem).wait()
    for bb in range(b):
        acc = s_ref[pl.ds(bb, 1), :]
        for k in range(_K):
            acc = acc + wscr[bb, k] * rows[pl.ds(bb * _K + k, 1), :]
        out_ref[pl.ds(bb, 1), :] = acc / wscr[bb, _K]


def kernel(x, W):
    B, T, D = x.shape
    n_chunks = T // _TC

    et, S = pl.pallas_call(
        _pass1_body,
        grid=(n_chunks,),
        in_specs=[
            pl.BlockSpec((B, _TC, D), lambda i: (0, i, 0)),
            pl.BlockSpec((D, 1), lambda i: (0, 0)),
        ],
        out_specs=[
            pl.BlockSpec((B, _TC), lambda i: (0, i)),
            pl.BlockSpec((B, D), lambda i: (0, 0)),
        ],
        out_shape=[
            jax.ShapeDtypeStruct((B, T), jnp.float32),
            jax.ShapeDtypeStruct((B, D), jnp.float32),
        ],
    )(x, W)

    out = pl.pallas_call(
        _tail_body,
        in_specs=[
            pl.BlockSpec((B, T), lambda: (0, 0)),
            pl.BlockSpec((B, D), lambda: (0, 0)),
            pl.BlockSpec(memory_space=pl.ANY),
        ],
        out_specs=pl.BlockSpec((B, D), lambda: (0, 0)),
        out_shape=jax.ShapeDtypeStruct((B, D), jnp.float32),
        scratch_shapes=[
            pltpu.VMEM((B, _PAD), jnp.int32),
            pltpu.VMEM((B, _PAD), jnp.float32),
            pltpu.VMEM((B * _K, D), jnp.float32),
            pltpu.SemaphoreType.DMA,
        ],
    )(et, S, x)

    return out


# speedup vs baseline: 1.4944x; 1.0764x over previous
import functools

import jax
import jax.numpy as jnp
from jax.experimental import pallas as pl
from jax.experimental.pallas import tpu as pltpu

_TC = 1024
_K = 8
_PAD = 128


def _pass1_body(x_ref, w_ref, et_ref, s_ref):
    i = pl.program_id(0)
    xb = x_ref[...]
    b, tc, d = xb.shape
    z = jax.lax.dot_general(
        xb.reshape(b * tc, d), w_ref[...],
        (((1,), (0,)), ((), ())),
        preferred_element_type=jnp.float32,
    )
    et_ref[...] = jnp.tanh(z).reshape(b, tc)
    part = jnp.sum(xb, axis=1)

    @pl.when(i == 0)
    def _():
        s_ref[...] = part

    @pl.when(i > 0)
    def _():
        s_ref[...] += part


def _topk_body(et_ref, s_ref, idx_ref, wz_ref, sz_ref):
    et = et_ref[...]
    b, t = et.shape
    iota = jax.lax.broadcasted_iota(jnp.int32, (b, t), 1)
    k_iota = jax.lax.broadcasted_iota(jnp.int32, (b, _PAD), 1)
    cur = et
    vals = jnp.full((b, _PAD), -2.0, jnp.float32)
    idxs = jnp.zeros((b, _PAD), jnp.int32)
    for k in range(_K):
        v = jnp.max(cur, axis=1, keepdims=True)
        am = jnp.min(jnp.where(cur == v, iota, t), axis=1, keepdims=True)
        vals = jnp.where(k_iota == k, v, vals)
        idxs = jnp.where(k_iota == k, am, idxs)
        cur = jnp.where(iota == am, -2.0, cur)
    thresh = jnp.sum(jnp.where(k_iota == 4, vals, 0.0), axis=1, keepdims=True)
    w = jnp.where(vals >= thresh, jnp.exp(vals) - 1.0, 0.0)
    zden = t + jnp.sum(w, axis=1, keepdims=True)
    idx_ref[...] = idxs
    wz_ref[...] = w / zden
    sz_ref[...] = s_ref[...] / zden


def _tail_body(et_ref, s_ref, x_hbm, out_ref, iscr, wscr, rows, sem):
    et = et_ref[...]
    b, t = et.shape
    iota = jax.lax.broadcasted_iota(jnp.int32, (b, t), 1)
    k_iota = jax.lax.broadcasted_iota(jnp.int32, (b, _PAD), 1)
    cur = et
    vals = jnp.full((b, _PAD), -2.0, jnp.float32)
    idxs = jnp.zeros((b, _PAD), jnp.int32)
    for k in range(_K):
        v = jnp.max(cur, axis=1, keepdims=True)
        am = jnp.min(jnp.where(cur == v, iota, t), axis=1, keepdims=True)
        vals = jnp.where(k_iota == k, v, vals)
        idxs = jnp.where(k_iota == k, am, idxs)
        cur = jnp.where(iota == am, -2.0, cur)
    thresh = jnp.sum(jnp.where(k_iota == 4, vals, 0.0), axis=1, keepdims=True)
    w = jnp.where(vals >= thresh, jnp.exp(vals) - 1.0, 0.0)
    zden = t + jnp.sum(w, axis=1, keepdims=True)
    iscr[...] = idxs
    wscr[...] = jnp.where(k_iota == _K, zden, w)
    for bb in range(b):
        for k in range(_K):
            pltpu.make_async_copy(
                x_hbm.at[bb, pl.ds(iscr[bb, k], 1), :],
                rows.at[pl.ds(bb * _K + k, 1), :], sem).start()
    for bb in range(b):
        for k in range(_K):
            pltpu.make_async_copy(
                x_hbm.at[bb, pl.ds(iscr[bb, k], 1), :],
                rows.at[pl.ds(bb * _K + k, 1), :], sem).wait()
    for bb in range(b):
        acc = s_ref[pl.ds(bb, 1), :]
        for k in range(_K):
            acc = acc + wscr[bb, k] * rows[pl.ds(bb * _K + k, 1), :]
        out_ref[pl.ds(bb, 1), :] = acc / wscr[bb, _K]


def kernel(x, W):
    B, T, D = x.shape
    n_chunks = T // _TC

    et, S = pl.pallas_call(
        _pass1_body,
        grid=(n_chunks,),
        in_specs=[
            pl.BlockSpec((B, _TC, D), lambda i: (0, i, 0)),
            pl.BlockSpec((D, 1), lambda i: (0, 0)),
        ],
        out_specs=[
            pl.BlockSpec((B, _TC), lambda i: (0, i)),
            pl.BlockSpec((B, D), lambda i: (0, 0)),
        ],
        out_shape=[
            jax.ShapeDtypeStruct((B, T), jnp.float32),
            jax.ShapeDtypeStruct((B, D), jnp.float32),
        ],
    )(x, W)

    out = pl.pallas_call(
        _tail_body,
        in_specs=[
            pl.BlockSpec((B, T), lambda: (0, 0)),
            pl.BlockSpec((B, D), lambda: (0, 0)),
            pl.BlockSpec(memory_space=pl.ANY),
        ],
        out_specs=pl.BlockSpec((B, D), lambda: (0, 0)),
        out_shape=jax.ShapeDtypeStruct((B, D), jnp.float32),
        scratch_shapes=[
            pltpu.VMEM((B, _PAD), jnp.int32),
            pltpu.VMEM((B, _PAD), jnp.float32),
            pltpu.VMEM((B * _K, D), jnp.float32),
            pltpu.SemaphoreType.DMA,
        ],
    )(et, S, x)

    return out
